# baseline (speedup 1.0000x reference)
import jax
import jax.numpy as jnp
from jax import lax
from jax.experimental import pallas as pl
from jax.experimental.pallas import tpu as pltpu
from jax.experimental.pallas import tpu_sc as plsc

N_CELLS = 100000
D = 128
B = 80
NBLK = N_CELLS // B
NW = 32
MAX_BLK_PER_W = (NBLK + NW - 1) // NW
LC = D // 16


def _sc_body(phi_hbm, i0_hbm, i1_hbm, i2_hbm, dist_hbm, area_hbm, unv_hbm,
             out_hbm,
             i0_v, i1_v, i2_v, d0_v, d1_v, a_v, w0_v, w1_v,
             g0_v, g1_v, g2_v, unv_v, sem):
    wid = lax.axis_index("s") * 2 + lax.axis_index("c")

    def block_body(t, _):
        k = wid + t * NW

        @pl.when(k < NBLK)
        def _():
            base = k * B
            pltpu.sync_copy(i0_hbm.at[pl.ds(base, B)], i0_v)
            pltpu.sync_copy(i1_hbm.at[pl.ds(base, B)], i1_v)
            pltpu.sync_copy(i2_hbm.at[pl.ds(base, B)], i2_v)
            c0 = pltpu.async_copy(phi_hbm.at[i0_v], g0_v, sem)
            c1 = pltpu.async_copy(phi_hbm.at[i1_v], g1_v, sem)
            c2 = pltpu.async_copy(phi_hbm.at[i2_v], g2_v, sem)
            c3 = pltpu.async_copy(dist_hbm.at[i0_v], d0_v, sem)
            c4 = pltpu.async_copy(dist_hbm.at[i1_v], d1_v, sem)
            pltpu.sync_copy(unv_hbm.at[pl.ds(base, B)], unv_v)
            pltpu.sync_copy(area_hbm.at[pl.ds(base, B)], a_v)
            c0.wait()
            c1.wait()
            c2.wait()
            c3.wait()
            c4.wait()

            for tt in range(B // 16):
                sl = pl.ds(tt * 16, 16)
                a = a_v[sl]
                w0_v[sl] = d0_v[sl] / a
                w1_v[sl] = d1_v[sl] / a

            def cell_body(b, _):
                w0 = jnp.full((16,), w0_v[pl.ds(b, 16)][0], jnp.float32)
                w1 = jnp.full((16,), w1_v[pl.ds(b, 16)][0], jnp.float32)
                for j in range(LC):
                    sl = pl.ds(j * 16, 16)
                    g0 = g0_v[b, sl]
                    g1 = g1_v[b, sl]
                    g2 = g2_v[b, sl]
                    u0 = unv_v[b, pl.ds(j * 16, 16)]
                    u1 = unv_v[b, pl.ds(D + j * 16, 16)]
                    u2 = unv_v[b, pl.ds(2 * D + j * 16, 16)]
                    g0_v[b, sl] = g0 * u0 * w0 + (g1 * u1 + g2 * u2) * w1
                return 0

            lax.fori_loop(0, B, cell_body, 0, unroll=False)

            pltpu.sync_copy(g0_v, out_hbm.at[pl.ds(base, B)])

        return 0

    lax.fori_loop(0, MAX_BLK_PER_W, block_body, 0, unroll=False)


def kernel(phi_f, cells_face, edge_Euclidean_distance, cell_area, unv,
           edge_neighbour_index, cells_type, face_type):
    del edge_neighbour_index, cells_type, face_type
    i0 = cells_face[0]
    i1 = cells_face[1]
    i2 = cells_face[2]
    dist = edge_Euclidean_distance.reshape(-1)
    area = cell_area.reshape(-1)
    unv2 = unv.reshape(N_CELLS, 3 * D)

    mesh = plsc.VectorSubcoreMesh(core_axis_name="c", subcore_axis_name="s")
    out = pl.kernel(
        _sc_body,
        out_type=jax.ShapeDtypeStruct((N_CELLS, D), jnp.float32),
        mesh=mesh,
        scratch_types=[
            pltpu.VMEM((B,), jnp.int32),
            pltpu.VMEM((B,), jnp.int32),
            pltpu.VMEM((B,), jnp.int32),
            pltpu.VMEM((B,), jnp.float32),
            pltpu.VMEM((B,), jnp.float32),
            pltpu.VMEM((B,), jnp.float32),
            pltpu.VMEM((B + 16,), jnp.float32),
            pltpu.VMEM((B + 16,), jnp.float32),
            pltpu.VMEM((B, D), jnp.float32),
            pltpu.VMEM((B, D), jnp.float32),
            pltpu.VMEM((B, D), jnp.float32),
            pltpu.VMEM((B, 3 * D), jnp.float32),
            pltpu.SemaphoreType.DMA,
        ],
    )(phi_f, i0, i1, i2, dist, area, unv2)
    return out
